# baseline (speedup 1.0000x reference)
import functools

import jax
import jax.numpy as jnp
from jax import lax
from jax.experimental import pallas as pl
from jax.experimental.pallas import tpu as pltpu
from jax.experimental.pallas import tpu_sc as plsc

_B, _N, _NBH, _D, _F, _G = 4, 2048, 64, 128, 128, 25
_CUTOFF = 5.0
_LN2 = 0.6931471805599453
_FH = _F // 2

_E = _B * _N * _NBH
_EBLK = 4096
_EH = _EBLK // 2
_ABLK = _EBLK // _NBH
_NCK = 8
_ECK = _E // _NCK
_NBLK = _ECK // _EBLK

_NC, _NS = 2, 16
_NW = _NC * _NS
_CHUNK = 128
_PER_W = _ECK // _NW
_NSTEP = _PER_W // _CHUNK


def _ssp(x):
    return jnp.maximum(x, 0.0) + jnp.log1p(jnp.exp(-jnp.abs(x))) - _LN2


def _in2f_body(x_ref, w_ref, y_ref):
    y = jnp.dot(x_ref[...], w_ref[...], preferred_element_type=jnp.float32)
    yb = y.astype(jnp.bfloat16)
    lo = lax.bitcast_convert_type(yb[:, :_FH], jnp.uint16).astype(jnp.uint32)
    hi = lax.bitcast_convert_type(yb[:, _FH:], jnp.uint16).astype(jnp.uint32)
    pk = lo | (hi << 16)
    y_ref[...] = lax.bitcast_convert_type(pk, jnp.int32)


_in2f = pl.pallas_call(
    _in2f_body,
    grid=(8,),
    in_specs=[
        pl.BlockSpec((_B * _N // 8, _D), lambda i: (i, 0)),
        pl.BlockSpec((_D, _F), lambda i: (0, 0)),
    ],
    out_specs=pl.BlockSpec((_B * _N // 8, _FH), lambda i: (i, 0)),
    out_shape=jax.ShapeDtypeStruct((_B * _N, _FH), jnp.int32),
)


def _gather_body(k, table_hbm, idx_hbm, out_hbm, idx_v, rows_v,
                 sem_g, sem_o0, sem_o1):
    wid = lax.axis_index("s") * _NC + lax.axis_index("c")
    base = wid * _PER_W
    qbase = wid * (_PER_W // 2)
    pltpu.sync_copy(idx_hbm.at[pl.ds(k * _ECK + base, _PER_W)], idx_v)

    def _wr(b, j):
        q = qbase + j * (_CHUNK // 2)
        yield (rows_v.at[b, pl.ds(0, _CHUNK // 2)],
               out_hbm.at[pl.ds(q, _CHUNK // 2), pl.ds(0, _FH)])
        yield (rows_v.at[b, pl.ds(_CHUNK // 2, _CHUNK // 2)],
               out_hbm.at[pl.ds(q, _CHUNK // 2), pl.ds(_FH, _FH)])

    def step(jj, carry):
        for b, sem_o in ((0, sem_o0), (1, sem_o1)):
            j = 2 * jj + b

            @pl.when(jj > 0)
            def _():
                for s, d in _wr(b, j - 2):
                    pltpu.make_async_copy(s, d, sem_o).wait()

            pltpu.async_copy(
                table_hbm.at[idx_v.at[pl.ds(j * _CHUNK, _CHUNK)]],
                rows_v.at[b], sem_g).wait()
            for s, d in _wr(b, j):
                pltpu.async_copy(s, d, sem_o)
        return carry

    lax.fori_loop(0, _NSTEP // 2, step, 0)
    for s, d in _wr(0, _NSTEP - 2):
        pltpu.make_async_copy(s, d, sem_o0).wait()
    for s, d in _wr(1, _NSTEP - 1):
        pltpu.make_async_copy(s, d, sem_o1).wait()


@functools.cache
def _make_gather(k):
    return pl.kernel(
        functools.partial(_gather_body, k),
        out_type=jax.ShapeDtypeStruct((_ECK // 2, _F), jnp.int32),
        compiler_params=pltpu.CompilerParams(use_tc_tiling_on_sc=False),
        mesh=plsc.VectorSubcoreMesh(core_axis_name="c", subcore_axis_name="s",
                                    num_cores=_NC, num_subcores=_NS),
        scratch_types=[
            pltpu.VMEM((_PER_W,), jnp.int32),
            pltpu.VMEM((2, _CHUNK, _FH), jnp.int32),
            pltpu.SemaphoreType.DMA,
            pltpu.SemaphoreType.DMA,
            pltpu.SemaphoreType.DMA,
        ],
        name=f"nbr_gather_{k}",
    )


def _conv_body(f_ref, ynb_ref, r_ref, nm_ref, w1_ref, b1_ref, w2_ref,
               b2_ref, wf2_ref, bf2_ref, wd_ref, bd_ref, o_ref):
    fb = f_ref[...]
    h = _ssp(jnp.dot(fb, w1_ref[...],
                     preferred_element_type=jnp.float32) + b1_ref[...])
    wf = jnp.dot(h.astype(jnp.bfloat16), w2_ref[...],
                 preferred_element_type=jnp.float32) + b2_ref[...]

    pk = ynb_ref[...]
    aw, bw = pk[:, :_FH], pk[:, _FH:]
    ya = jnp.concatenate(
        [lax.bitcast_convert_type(aw << 16, jnp.float32),
         lax.bitcast_convert_type(aw & jnp.int32(-65536), jnp.float32)],
        axis=1)
    yb = jnp.concatenate(
        [lax.bitcast_convert_type(bw << 16, jnp.float32),
         lax.bitcast_convert_type(bw & jnp.int32(-65536), jnp.float32)],
        axis=1)
    ynb = jnp.concatenate([ya, yb], axis=0)
    p = wf * ynb

    m = (r_ref[0] <= _CUTOFF).astype(jnp.float32) * nm_ref[0]
    a_io = lax.broadcasted_iota(jnp.int32, (_ABLK, _EBLK), 0)
    c_io = lax.broadcasted_iota(jnp.int32, (_ABLK, _EBLK), 1)
    sel = (c_io // _NBH) == a_io
    mm = jnp.where(sel, jnp.broadcast_to(m, (_ABLK, _EBLK)), 0.0)
    agg = jnp.dot(mm.astype(jnp.bfloat16), p.astype(jnp.bfloat16),
                  preferred_element_type=jnp.float32)

    v = _ssp(jnp.dot(agg, wf2_ref[...],
                     preferred_element_type=jnp.float32) + bf2_ref[...])
    o_ref[...] = jnp.dot(v, wd_ref[...],
                         preferred_element_type=jnp.float32) + bd_ref[...]


@functools.cache
def _make_conv(k):
    blk = _NBLK * k

    return pl.pallas_call(
        _conv_body,
        grid=(_NBLK,),
        in_specs=[
            pl.BlockSpec((_EBLK, _G), lambda i: (blk + i, 0)),
            pl.BlockSpec((_EH, _F), lambda i: (i, 0)),
            pl.BlockSpec((1, 1, _EBLK), lambda i: (blk + i, 0, 0)),
            pl.BlockSpec((1, 1, _EBLK), lambda i: (blk + i, 0, 0)),
            pl.BlockSpec((_G, _F), lambda i: (0, 0)),
            pl.BlockSpec((1, _F), lambda i: (0, 0)),
            pl.BlockSpec((_F, _F), lambda i: (0, 0)),
            pl.BlockSpec((1, _F), lambda i: (0, 0)),
            pl.BlockSpec((_F, _D), lambda i: (0, 0)),
            pl.BlockSpec((1, _D), lambda i: (0, 0)),
            pl.BlockSpec((_D, _D), lambda i: (0, 0)),
            pl.BlockSpec((1, _D), lambda i: (0, 0)),
        ],
        out_specs=pl.BlockSpec((_ABLK, _D), lambda i: (i, 0)),
        out_shape=jax.ShapeDtypeStruct((_ECK // _NBH, _D), jnp.float32),
        name=f"cfconv_{k}",
    )


def kernel(x, r_ij, neighbors, neighbor_mask, f_ij, W1, b1, W2, b2, W_in,
           W_f2, b_f2, W_d, b_d):
    x2 = x.reshape(_B * _N, _D)
    y = _in2f(x2, W_in)

    offs = (jnp.arange(_B, dtype=jnp.int32) * _N)[:, None, None]
    idx = (neighbors.astype(jnp.int32) + offs).reshape(_E)
    idx4 = idx.reshape(_NCK, _NBLK, 2, _EH)
    a = idx4[:, :, 0, :].reshape(_NCK, -1, _CHUNK // 2)
    b = idx4[:, :, 1, :].reshape(_NCK, -1, _CHUNK // 2)
    idx = jnp.stack([a, b], axis=2).reshape(_E)

    f2 = f_ij.reshape(_E, _G).astype(jnp.bfloat16)
    r3 = r_ij.reshape(_E // _EBLK, 1, _EBLK)
    nm3 = neighbor_mask.reshape(_E // _EBLK, 1, _EBLK)
    w = (b1.reshape(1, _F), W2.astype(jnp.bfloat16), b2.reshape(1, _F),
         W_f2, b_f2.reshape(1, _D), W_d, b_d.reshape(1, _D))

    w1b = W1.astype(jnp.bfloat16)
    outs = []
    for k in range(_NCK):
        y_nb = _make_gather(k)(y, idx)
        outs.append(_make_conv(k)(f2, y_nb, r3, nm3, w1b, *w))
    v = jnp.concatenate(outs, axis=0)
    return v.reshape(_B, _N, _D)


# speedup vs baseline: 1.1110x; 1.1110x over previous
import functools

import jax
import jax.numpy as jnp
from jax import lax
from jax.experimental import pallas as pl
from jax.experimental.pallas import tpu as pltpu
from jax.experimental.pallas import tpu_sc as plsc

_B, _N, _NBH, _D, _F, _G = 4, 2048, 64, 128, 128, 25
_CUTOFF = 5.0
_LN2 = 0.6931471805599453
_FH = _F // 2

_E = _B * _N * _NBH
_EBLK = 4096
_EH = _EBLK // 2
_ABLK = _EBLK // _NBH
_NCK = 8
_ECK = _E // _NCK
_NBLK = _ECK // _EBLK

_NC, _NS = 2, 16
_NW = _NC * _NS
_CHUNK = 128
_PER_W = _ECK // _NW
_NSTEP = _PER_W // _CHUNK


_LOG2E = 1.4426950408889634


def _ssp(x):
    return _LN2 * (jnp.log2(1.0 + jnp.exp2(x * _LOG2E)) - 1.0)


def _in2f_body(x_ref, w_ref, y_ref):
    y = jnp.dot(x_ref[...], w_ref[...], preferred_element_type=jnp.float32)
    yb = y.astype(jnp.bfloat16)
    lo = lax.bitcast_convert_type(yb[:, :_FH], jnp.uint16).astype(jnp.uint32)
    hi = lax.bitcast_convert_type(yb[:, _FH:], jnp.uint16).astype(jnp.uint32)
    pk = lo | (hi << 16)
    y_ref[...] = lax.bitcast_convert_type(pk, jnp.int32)


_in2f = pl.pallas_call(
    _in2f_body,
    grid=(8,),
    in_specs=[
        pl.BlockSpec((_B * _N // 8, _D), lambda i: (i, 0)),
        pl.BlockSpec((_D, _F), lambda i: (0, 0)),
    ],
    out_specs=pl.BlockSpec((_B * _N // 8, _FH), lambda i: (i, 0)),
    out_shape=jax.ShapeDtypeStruct((_B * _N, _FH), jnp.int32),
)


def _gather_body(k, table_hbm, idx_hbm, out_hbm, idx_v, rows_v,
                 sem_g, sem_o0, sem_o1):
    wid = lax.axis_index("s") * _NC + lax.axis_index("c")
    base = wid * _PER_W
    qbase = wid * (_PER_W // 2)
    pltpu.sync_copy(idx_hbm.at[pl.ds(k * _ECK + base, _PER_W)], idx_v)

    def _wr(b, j):
        q = qbase + j * (_CHUNK // 2)
        yield (rows_v.at[b, pl.ds(0, _CHUNK // 2)],
               out_hbm.at[pl.ds(q, _CHUNK // 2), pl.ds(0, _FH)])
        yield (rows_v.at[b, pl.ds(_CHUNK // 2, _CHUNK // 2)],
               out_hbm.at[pl.ds(q, _CHUNK // 2), pl.ds(_FH, _FH)])

    def step(jj, carry):
        for b, sem_o in ((0, sem_o0), (1, sem_o1)):
            j = 2 * jj + b

            @pl.when(jj > 0)
            def _():
                for s, d in _wr(b, j - 2):
                    pltpu.make_async_copy(s, d, sem_o).wait()

            pltpu.async_copy(
                table_hbm.at[idx_v.at[pl.ds(j * _CHUNK, _CHUNK)]],
                rows_v.at[b], sem_g).wait()
            for s, d in _wr(b, j):
                pltpu.async_copy(s, d, sem_o)
        return carry

    lax.fori_loop(0, _NSTEP // 2, step, 0)
    for s, d in _wr(0, _NSTEP - 2):
        pltpu.make_async_copy(s, d, sem_o0).wait()
    for s, d in _wr(1, _NSTEP - 1):
        pltpu.make_async_copy(s, d, sem_o1).wait()


@functools.cache
def _make_gather(k):
    return pl.kernel(
        functools.partial(_gather_body, k),
        out_type=jax.ShapeDtypeStruct((_ECK // 2, _F), jnp.int32),
        compiler_params=pltpu.CompilerParams(use_tc_tiling_on_sc=False),
        mesh=plsc.VectorSubcoreMesh(core_axis_name="c", subcore_axis_name="s",
                                    num_cores=_NC, num_subcores=_NS),
        scratch_types=[
            pltpu.VMEM((_PER_W,), jnp.int32),
            pltpu.VMEM((2, _CHUNK, _FH), jnp.int32),
            pltpu.SemaphoreType.DMA,
            pltpu.SemaphoreType.DMA,
            pltpu.SemaphoreType.DMA,
        ],
        name=f"nbr_gather_{k}",
    )


def _conv_body(f_ref, ynb_ref, r_ref, nm_ref, w1_ref, b1_ref, w2_ref,
               b2_ref, wf2_ref, bf2_ref, wd_ref, bd_ref, o_ref):
    fb = f_ref[...]
    h = _ssp(jnp.dot(fb, w1_ref[...],
                     preferred_element_type=jnp.float32) + b1_ref[...])
    wf = jnp.dot(h, w2_ref[...],
                 preferred_element_type=jnp.float32) + b2_ref[...]

    pk = ynb_ref[...]
    aw, bw = pk[:, :_FH], pk[:, _FH:]
    ya = jnp.concatenate(
        [lax.bitcast_convert_type(aw << 16, jnp.float32),
         lax.bitcast_convert_type(aw & jnp.int32(-65536), jnp.float32)],
        axis=1)
    yb = jnp.concatenate(
        [lax.bitcast_convert_type(bw << 16, jnp.float32),
         lax.bitcast_convert_type(bw & jnp.int32(-65536), jnp.float32)],
        axis=1)
    ynb = jnp.concatenate([ya, yb], axis=0)
    p = wf * ynb

    m = (r_ref[0] <= _CUTOFF).astype(jnp.float32) * nm_ref[0]
    a_io = lax.broadcasted_iota(jnp.int32, (_ABLK, _EBLK), 0)
    c_io = lax.broadcasted_iota(jnp.int32, (_ABLK, _EBLK), 1)
    sel = (c_io // _NBH) == a_io
    mm = jnp.where(sel, jnp.broadcast_to(m, (_ABLK, _EBLK)), 0.0)
    agg = jnp.dot(mm, p, preferred_element_type=jnp.float32)

    v = _ssp(jnp.dot(agg, wf2_ref[...],
                     preferred_element_type=jnp.float32) + bf2_ref[...])
    o_ref[...] = jnp.dot(v, wd_ref[...],
                         preferred_element_type=jnp.float32) + bd_ref[...]


@functools.cache
def _make_conv(k):
    blk = _NBLK * k

    return pl.pallas_call(
        _conv_body,
        grid=(_NBLK,),
        in_specs=[
            pl.BlockSpec((_EBLK, _G), lambda i: (blk + i, 0)),
            pl.BlockSpec((_EH, _F), lambda i: (i, 0)),
            pl.BlockSpec((1, 1, _EBLK), lambda i: (blk + i, 0, 0)),
            pl.BlockSpec((1, 1, _EBLK), lambda i: (blk + i, 0, 0)),
            pl.BlockSpec((_G, _F), lambda i: (0, 0)),
            pl.BlockSpec((1, _F), lambda i: (0, 0)),
            pl.BlockSpec((_F, _F), lambda i: (0, 0)),
            pl.BlockSpec((1, _F), lambda i: (0, 0)),
            pl.BlockSpec((_F, _D), lambda i: (0, 0)),
            pl.BlockSpec((1, _D), lambda i: (0, 0)),
            pl.BlockSpec((_D, _D), lambda i: (0, 0)),
            pl.BlockSpec((1, _D), lambda i: (0, 0)),
        ],
        out_specs=pl.BlockSpec((_ABLK, _D), lambda i: (i, 0)),
        out_shape=jax.ShapeDtypeStruct((_ECK // _NBH, _D), jnp.float32),
        name=f"cfconv_{k}",
    )


def kernel(x, r_ij, neighbors, neighbor_mask, f_ij, W1, b1, W2, b2, W_in,
           W_f2, b_f2, W_d, b_d):
    x2 = x.reshape(_B * _N, _D)
    y = _in2f(x2, W_in)

    offs = (jnp.arange(_B, dtype=jnp.int32) * _N)[:, None, None]
    idx = (neighbors.astype(jnp.int32) + offs).reshape(_E)
    idx4 = idx.reshape(_NCK, _NBLK, 2, _EH)
    a = idx4[:, :, 0, :].reshape(_NCK, -1, _CHUNK // 2)
    b = idx4[:, :, 1, :].reshape(_NCK, -1, _CHUNK // 2)
    idx = jnp.stack([a, b], axis=2).reshape(_E)

    f2 = f_ij.reshape(_E, _G).astype(jnp.bfloat16)
    r3 = r_ij.reshape(_E // _EBLK, 1, _EBLK)
    nm3 = neighbor_mask.reshape(_E // _EBLK, 1, _EBLK)
    w = (b1.reshape(1, _F), W2, b2.reshape(1, _F),
         W_f2, b_f2.reshape(1, _D), W_d, b_d.reshape(1, _D))

    w1b = W1.astype(jnp.bfloat16)
    outs = []
    for k in range(_NCK):
        y_nb = _make_gather(k)(y, idx)
        outs.append(_make_conv(k)(f2, y_nb, r3, nm3, w1b, *w))
    v = jnp.concatenate(outs, axis=0)
    return v.reshape(_B, _N, _D)
